# baseline (speedup 1.0000x reference)
import functools

import jax
import jax.numpy as jnp
import numpy as np
from jax import lax
from jax.experimental import pallas as pl
from jax.experimental.pallas import tpu as pltpu
from jax.experimental.pallas import tpu_sc as plsc

_N_WORDS = 8192
_N_SAMP = 65536
_WORD_DIM = 256
_NC = 2
_NS = 16
_CH = 16
_K = 8
_NBUF = 4
_XG = 112
_HALF = _N_WORDS // _NC


def _host_idx():
    cpu = jax.devices("cpu")[0]
    with jax.default_device(cpu):
        perm = jax.random.permutation(jax.random.key(1), _N_SAMP)
        return np.asarray(jax.device_get(perm))[:_N_WORDS].astype(np.int64)


_IDX_NP = _host_idx()


def _build_meta(idx_np):
    midx = np.zeros((_NC * _NS, _XG), np.int32)
    chunks = [{} for _ in range(_NC)]
    for i, src in enumerate(idx_np.tolist()):
        c = i // _HALF
        off = i % (_HALF // _NS)
        if off < _XG:
            midx[i // (_HALF // _NS), off] = src
            continue
        chunks[c].setdefault(src // _CH, []).append((src % _CH, i - c * _HALF))
    assign = {}
    max_ne = 0
    for c in range(_NC):
        order = sorted(chunks[c].items(), key=lambda kv: -len(kv[1]))
        loads = [(0, 0, t) for t in range(_NS)]
        lists = [[] for _ in range(_NS)]
        for cid, ent in order:
            assert len(ent) <= _K
            loads.sort()
            rows, ne, t = loads[0]
            lists[t].append((cid, ent))
            loads[0] = (rows + len(ent), ne + 1, t)
        for t in range(_NS):
            assign[c * _NS + t] = lists[t]
            max_ne = max(max_ne, len(lists[t]))
    lp = 2 + 2 * max_ne + 32
    ls = _K * max_ne + 32
    mpack = np.zeros((_NC * _NS, lp), np.int32)
    msrc = np.zeros((_NC * _NS, ls), np.int32)
    mdst = np.zeros((_NC * _NS, ls), np.int32)
    for w, lst in assign.items():
        mpack[w, 0] = len(lst)
        for e, (cid, ent) in enumerate(lst):
            mpack[w, 2 + 2 * e] = cid
            mpack[w, 3 + 2 * e] = len(ent)
            for j, (sl, dl) in enumerate(ent):
                msrc[w, _K * e + j] = sl
                mdst[w, _K * e + j] = dl
    return mpack, msrc, mdst, midx


def _sc_scan_route(table, mpack, msrc, mdst, midx):
    lp, ls = mpack.shape[1], msrc.shape[1]
    mesh = plsc.VectorSubcoreMesh(core_axis_name="c", subcore_axis_name="s")

    @functools.partial(
        pl.kernel,
        mesh=mesh,
        out_type=jax.ShapeDtypeStruct((_N_WORDS, _WORD_DIM), jnp.float32),
        scratch_types=[
            pltpu.VMEM((lp,), jnp.int32),
            pltpu.VMEM((ls,), jnp.int32),
            pltpu.VMEM((ls,), jnp.int32),
            pltpu.VMEM((_NBUF, _CH, _WORD_DIM), jnp.float32),
            pltpu.VMEM((1, _WORD_DIM), jnp.float32),
            pltpu.VMEM((_XG,), jnp.int32),
            pltpu.VMEM((_XG, _WORD_DIM), jnp.float32),
            pltpu.VMEM_SHARED((_HALF, _WORD_DIM), jnp.float32),
        ]
        + [pltpu.SemaphoreType.DMA] * (2 * _NBUF + 2),
    )
    def k(table_h, mpack_h, msrc_h, mdst_h, midx_h, out_h,
          mpack_v, msrc_v, mdst_v, bufs, drainbuf, midx_v, gbuf, spmem,
          *sems):
        ssem = sems[:_NBUF]
        rowsem = sems[_NBUF:2 * _NBUF]
        gsem = sems[2 * _NBUF]
        msem = sems[2 * _NBUF + 1]
        c = lax.axis_index("c")
        s = lax.axis_index("s")
        w = c * _NS + s
        mcopies = [
            pltpu.async_copy(midx_h.at[w], midx_v, msem),
            pltpu.async_copy(mpack_h.at[w], mpack_v, msem),
            pltpu.async_copy(msrc_h.at[w], msrc_v, msem),
            pltpu.async_copy(mdst_h.at[w], mdst_v, msem),
        ]
        mcopies[0].wait()
        gcopy = pltpu.async_copy(table_h.at[midx_v], gbuf, gsem)
        for m in mcopies[1:]:
            m.wait()
        ne = mpack_v[pl.ds(0, 16)][0]

        def start(e, b):
            cid = mpack_v[pl.ds(2 + 2 * e, 16)][0]
            pltpu.async_copy(
                table_h.at[pl.ds(cid * _CH, _CH)], bufs.at[b], ssem[b]
            )

        for b in range(_NBUF - 1):
            @pl.when(b < ne)
            def _(b=b):
                start(b, b)

        @pl.loop(0, (ne + _NBUF) // _NBUF)
        def _(t):
            for b in range(_NBUF):
                g = t * _NBUF + b

                @pl.when(g < ne)
                def _(g=g, b=b):
                    pltpu.make_async_copy(
                        table_h.at[pl.ds(0, _CH)], bufs.at[b], ssem[b]
                    ).wait()
                    cnt = mpack_v[pl.ds(2 + 2 * g, 16)][1]
                    srow = msrc_v[pl.ds(_K * g, 16)]
                    drow = mdst_v[pl.ds(_K * g, 16)]
                    for j in range(_K):
                        @pl.when(j < cnt)
                        def _(j=j):
                            pltpu.async_copy(
                                bufs.at[b].at[pl.ds(srow[j], 1)],
                                spmem.at[pl.ds(drow[j], 1)],
                                rowsem[b],
                            )

                @pl.when((g >= 1) & (g <= ne))
                def _(g=g, b=b):
                    cp = mpack_v[pl.ds(2 * g, 16)][1]
                    for j in range(_K):
                        @pl.when(j < cp)
                        def _(j=j):
                            pltpu.make_async_copy(
                                table_h.at[pl.ds(0, 1)],
                                drainbuf,
                                rowsem[(b + _NBUF - 1) % _NBUF],
                            ).wait()

                @pl.when(g + (_NBUF - 1) < ne)
                def _(g=g, b=b):
                    start(g + (_NBUF - 1), (b + _NBUF - 1) % _NBUF)

        gcopy.wait()
        base = c * _HALF + s * (_HALF // _NS)
        pltpu.sync_copy(gbuf, out_h.at[pl.ds(base, _XG)])
        plsc.subcore_barrier()
        pltpu.sync_copy(
            spmem.at[pl.ds(s * (_HALF // _NS) + _XG,
                           _HALF // _NS - _XG)],
            out_h.at[pl.ds(base + _XG, _HALF // _NS - _XG)],
        )

    return k(table, mpack, msrc, mdst, midx)


def kernel(z):
    mpack, msrc, mdst, midx = _build_meta(_IDX_NP)
    return _sc_scan_route(z, mpack, msrc, mdst, midx)


# speedup vs baseline: 1.1045x; 1.1045x over previous
import functools

import jax
import jax.numpy as jnp
import numpy as np
from jax import lax
from jax.experimental import pallas as pl
from jax.experimental.pallas import tpu as pltpu
from jax.experimental.pallas import tpu_sc as plsc

_N_WORDS = 8192
_N_SAMP = 65536
_WORD_DIM = 256
_NC = 2
_NS = 16
_CH = 16
_K = 8
_NBUF = 4
_XG = 144
_HALF = _N_WORDS // _NC


def _host_idx():
    cpu = jax.devices("cpu")[0]
    with jax.default_device(cpu):
        perm = jax.random.permutation(jax.random.key(1), _N_SAMP)
        return np.asarray(jax.device_get(perm))[:_N_WORDS].astype(np.int64)


_IDX_NP = _host_idx()


def _build_meta(idx_np):
    midx = np.zeros((_NC * _NS, _XG), np.int32)
    chunks = [{} for _ in range(_NC)]
    for i, src in enumerate(idx_np.tolist()):
        c = i // _HALF
        off = i % (_HALF // _NS)
        if off < _XG:
            midx[i // (_HALF // _NS), off] = src
            continue
        chunks[c].setdefault(src // _CH, []).append((src % _CH, i - c * _HALF))
    assign = {}
    max_ne = 0
    for c in range(_NC):
        order = sorted(chunks[c].items(), key=lambda kv: -len(kv[1]))
        loads = [(0, 0, t) for t in range(_NS)]
        lists = [[] for _ in range(_NS)]
        for cid, ent in order:
            assert len(ent) <= _K
            loads.sort()
            rows, ne, t = loads[0]
            lists[t].append((cid, ent))
            loads[0] = (rows + len(ent), ne + 1, t)
        for t in range(_NS):
            assign[c * _NS + t] = lists[t]
            max_ne = max(max_ne, len(lists[t]))
    lp = 2 + 2 * max_ne + 32
    ls = _K * max_ne + 32
    mpack = np.zeros((_NC * _NS, lp), np.int32)
    msrc = np.zeros((_NC * _NS, ls), np.int32)
    mdst = np.zeros((_NC * _NS, ls), np.int32)
    for w, lst in assign.items():
        mpack[w, 0] = len(lst)
        for e, (cid, ent) in enumerate(lst):
            mpack[w, 2 + 2 * e] = cid
            mpack[w, 3 + 2 * e] = len(ent)
            for j, (sl, dl) in enumerate(ent):
                msrc[w, _K * e + j] = sl
                mdst[w, _K * e + j] = dl
    return mpack, msrc, mdst, midx


def _sc_scan_route(table, mpack, msrc, mdst, midx):
    lp, ls = mpack.shape[1], msrc.shape[1]
    mesh = plsc.VectorSubcoreMesh(core_axis_name="c", subcore_axis_name="s")

    @functools.partial(
        pl.kernel,
        mesh=mesh,
        out_type=jax.ShapeDtypeStruct((_N_WORDS, _WORD_DIM), jnp.float32),
        scratch_types=[
            pltpu.VMEM((lp,), jnp.int32),
            pltpu.VMEM((ls,), jnp.int32),
            pltpu.VMEM((ls,), jnp.int32),
            pltpu.VMEM((_NBUF, _CH, _WORD_DIM), jnp.float32),
            pltpu.VMEM((1, _WORD_DIM), jnp.float32),
            pltpu.VMEM((_XG,), jnp.int32),
            pltpu.VMEM((_XG, _WORD_DIM), jnp.float32),
            pltpu.VMEM_SHARED((_HALF, _WORD_DIM), jnp.float32),
        ]
        + [pltpu.SemaphoreType.DMA] * (2 * _NBUF + 2),
    )
    def k(table_h, mpack_h, msrc_h, mdst_h, midx_h, out_h,
          mpack_v, msrc_v, mdst_v, bufs, drainbuf, midx_v, gbuf, spmem,
          *sems):
        ssem = sems[:_NBUF]
        rowsem = sems[_NBUF:2 * _NBUF]
        gsem = sems[2 * _NBUF]
        msem = sems[2 * _NBUF + 1]
        c = lax.axis_index("c")
        s = lax.axis_index("s")
        w = c * _NS + s
        mcopies = [
            pltpu.async_copy(midx_h.at[w], midx_v, msem),
            pltpu.async_copy(mpack_h.at[w], mpack_v, msem),
            pltpu.async_copy(msrc_h.at[w], msrc_v, msem),
            pltpu.async_copy(mdst_h.at[w], mdst_v, msem),
        ]
        mcopies[0].wait()
        gcopies = [
            pltpu.async_copy(
                table_h.at[midx_v.at[pl.ds(0, 128)]],
                gbuf.at[pl.ds(0, 128)], gsem),
            pltpu.async_copy(
                table_h.at[midx_v.at[pl.ds(128, _XG - 128)]],
                gbuf.at[pl.ds(128, _XG - 128)], gsem),
        ]
        for m in mcopies[1:]:
            m.wait()
        ne = mpack_v[pl.ds(0, 16)][0]

        def start(e, b):
            cid = mpack_v[pl.ds(2 + 2 * e, 16)][0]
            pltpu.async_copy(
                table_h.at[pl.ds(cid * _CH, _CH)], bufs.at[b], ssem[b]
            )

        for b in range(_NBUF - 1):
            @pl.when(b < ne)
            def _(b=b):
                start(b, b)

        @pl.loop(0, (ne + _NBUF) // _NBUF)
        def _(t):
            for b in range(_NBUF):
                g = t * _NBUF + b

                @pl.when(g < ne)
                def _(g=g, b=b):
                    pltpu.make_async_copy(
                        table_h.at[pl.ds(0, _CH)], bufs.at[b], ssem[b]
                    ).wait()
                    cnt = mpack_v[pl.ds(2 + 2 * g, 16)][1]
                    srow = msrc_v[pl.ds(_K * g, 16)]
                    drow = mdst_v[pl.ds(_K * g, 16)]
                    for j in range(_K):
                        @pl.when(j < cnt)
                        def _(j=j):
                            pltpu.async_copy(
                                bufs.at[b].at[pl.ds(srow[j], 1)],
                                spmem.at[pl.ds(drow[j], 1)],
                                rowsem[b],
                            )

                @pl.when((g >= 1) & (g <= ne))
                def _(g=g, b=b):
                    cp = mpack_v[pl.ds(2 * g, 16)][1]
                    for j in range(_K):
                        @pl.when(j < cp)
                        def _(j=j):
                            pltpu.make_async_copy(
                                table_h.at[pl.ds(0, 1)],
                                drainbuf,
                                rowsem[(b + _NBUF - 1) % _NBUF],
                            ).wait()

                @pl.when(g + (_NBUF - 1) < ne)
                def _(g=g, b=b):
                    start(g + (_NBUF - 1), (b + _NBUF - 1) % _NBUF)

        for gc in gcopies:
            gc.wait()
        base = c * _HALF + s * (_HALF // _NS)
        pltpu.sync_copy(gbuf, out_h.at[pl.ds(base, _XG)])
        plsc.subcore_barrier()
        pltpu.sync_copy(
            spmem.at[pl.ds(s * (_HALF // _NS) + _XG,
                           _HALF // _NS - _XG)],
            out_h.at[pl.ds(base + _XG, _HALF // _NS - _XG)],
        )

    return k(table, mpack, msrc, mdst, midx)


def kernel(z):
    mpack, msrc, mdst, midx = _build_meta(_IDX_NP)
    return _sc_scan_route(z, mpack, msrc, mdst, midx)


# speedup vs baseline: 1.1779x; 1.0665x over previous
import functools

import jax
import jax.numpy as jnp
import numpy as np
from jax import lax
from jax.experimental import pallas as pl
from jax.experimental.pallas import tpu as pltpu
from jax.experimental.pallas import tpu_sc as plsc

_N_WORDS = 8192
_N_SAMP = 65536
_WORD_DIM = 256
_NC = 2
_NS = 16
_CH = 16
_K = 8
_NBUF = 4
_XG = 160
_HALF = _N_WORDS // _NC


def _host_idx():
    cpu = jax.devices("cpu")[0]
    with jax.default_device(cpu):
        perm = jax.random.permutation(jax.random.key(1), _N_SAMP)
        return np.asarray(jax.device_get(perm))[:_N_WORDS].astype(np.int64)


_IDX_NP = _host_idx()


def _build_meta(idx_np):
    midx = np.zeros((_NC * _NS, _XG), np.int32)
    chunks = [{} for _ in range(_NC)]
    for i, src in enumerate(idx_np.tolist()):
        c = i // _HALF
        off = i % (_HALF // _NS)
        if off < _XG:
            midx[i // (_HALF // _NS), off] = src
            continue
        chunks[c].setdefault(src // _CH, []).append((src % _CH, i - c * _HALF))
    assign = {}
    max_ne = 0
    for c in range(_NC):
        order = sorted(chunks[c].items(), key=lambda kv: -len(kv[1]))
        loads = [(0, 0, t) for t in range(_NS)]
        lists = [[] for _ in range(_NS)]
        for cid, ent in order:
            assert len(ent) <= _K
            loads.sort()
            rows, ne, t = loads[0]
            lists[t].append((cid, ent))
            loads[0] = (rows + len(ent), ne + 1, t)
        for t in range(_NS):
            assign[c * _NS + t] = lists[t]
            max_ne = max(max_ne, len(lists[t]))
    lp = 2 + 2 * max_ne + 32
    ls = _K * max_ne + 32
    mpack = np.zeros((_NC * _NS, lp), np.int32)
    msrc = np.zeros((_NC * _NS, ls), np.int32)
    mdst = np.zeros((_NC * _NS, ls), np.int32)
    for w, lst in assign.items():
        mpack[w, 0] = len(lst)
        for e, (cid, ent) in enumerate(lst):
            mpack[w, 2 + 2 * e] = cid
            mpack[w, 3 + 2 * e] = len(ent)
            for j, (sl, dl) in enumerate(ent):
                msrc[w, _K * e + j] = sl
                mdst[w, _K * e + j] = dl
    return mpack, msrc, mdst, midx


def _sc_scan_route(table, mpack, msrc, mdst, midx):
    lp, ls = mpack.shape[1], msrc.shape[1]
    mesh = plsc.VectorSubcoreMesh(core_axis_name="c", subcore_axis_name="s")

    @functools.partial(
        pl.kernel,
        mesh=mesh,
        out_type=jax.ShapeDtypeStruct((_N_WORDS, _WORD_DIM), jnp.float32),
        scratch_types=[
            pltpu.VMEM((lp,), jnp.int32),
            pltpu.VMEM((ls,), jnp.int32),
            pltpu.VMEM((ls,), jnp.int32),
            pltpu.VMEM((_NBUF, _CH, _WORD_DIM), jnp.float32),
            pltpu.VMEM((1, _WORD_DIM), jnp.float32),
            pltpu.VMEM((_XG,), jnp.int32),
            pltpu.VMEM((_XG, _WORD_DIM), jnp.float32),
            pltpu.VMEM_SHARED((_HALF, _WORD_DIM), jnp.float32),
        ]
        + [pltpu.SemaphoreType.DMA] * (2 * _NBUF + 2),
    )
    def k(table_h, mpack_h, msrc_h, mdst_h, midx_h, out_h,
          mpack_v, msrc_v, mdst_v, bufs, drainbuf, midx_v, gbuf, spmem,
          *sems):
        ssem = sems[:_NBUF]
        rowsem = sems[_NBUF:2 * _NBUF]
        gsem = sems[2 * _NBUF]
        msem = sems[2 * _NBUF + 1]
        c = lax.axis_index("c")
        s = lax.axis_index("s")
        w = c * _NS + s
        mcopies = [
            pltpu.async_copy(midx_h.at[w], midx_v, msem),
            pltpu.async_copy(mpack_h.at[w], mpack_v, msem),
            pltpu.async_copy(msrc_h.at[w], msrc_v, msem),
            pltpu.async_copy(mdst_h.at[w], mdst_v, msem),
        ]
        mcopies[0].wait()
        gcopies = [
            pltpu.async_copy(
                table_h.at[midx_v.at[pl.ds(0, 128)]],
                gbuf.at[pl.ds(0, 128)], gsem),
            pltpu.async_copy(
                table_h.at[midx_v.at[pl.ds(128, _XG - 128)]],
                gbuf.at[pl.ds(128, _XG - 128)], gsem),
        ]
        for m in mcopies[1:]:
            m.wait()
        ne = mpack_v[pl.ds(0, 16)][0]

        def start(e, b):
            cid = mpack_v[pl.ds(2 + 2 * e, 16)][0]
            pltpu.async_copy(
                table_h.at[pl.ds(cid * _CH, _CH)], bufs.at[b], ssem[b]
            )

        for b in range(_NBUF - 1):
            @pl.when(b < ne)
            def _(b=b):
                start(b, b)

        @pl.loop(0, (ne + _NBUF) // _NBUF)
        def _(t):
            for b in range(_NBUF):
                g = t * _NBUF + b

                @pl.when(g < ne)
                def _(g=g, b=b):
                    pltpu.make_async_copy(
                        table_h.at[pl.ds(0, _CH)], bufs.at[b], ssem[b]
                    ).wait()
                    cnt = mpack_v[pl.ds(2 + 2 * g, 16)][1]
                    srow = msrc_v[pl.ds(_K * g, 16)]
                    drow = mdst_v[pl.ds(_K * g, 16)]
                    for j in range(_K):
                        @pl.when(j < cnt)
                        def _(j=j):
                            pltpu.async_copy(
                                bufs.at[b].at[pl.ds(srow[j], 1)],
                                spmem.at[pl.ds(drow[j], 1)],
                                rowsem[b],
                            )

                @pl.when((g >= 1) & (g <= ne))
                def _(g=g, b=b):
                    cp = mpack_v[pl.ds(2 * g, 16)][1]
                    for j in range(_K):
                        @pl.when(j < cp)
                        def _(j=j):
                            pltpu.make_async_copy(
                                table_h.at[pl.ds(0, 1)],
                                drainbuf,
                                rowsem[(b + _NBUF - 1) % _NBUF],
                            ).wait()

                @pl.when(g + (_NBUF - 1) < ne)
                def _(g=g, b=b):
                    start(g + (_NBUF - 1), (b + _NBUF - 1) % _NBUF)

        for gc in gcopies:
            gc.wait()
        base = c * _HALF + s * (_HALF // _NS)
        pltpu.sync_copy(gbuf, out_h.at[pl.ds(base, _XG)])
        plsc.subcore_barrier()
        pltpu.sync_copy(
            spmem.at[pl.ds(s * (_HALF // _NS) + _XG,
                           _HALF // _NS - _XG)],
            out_h.at[pl.ds(base + _XG, _HALF // _NS - _XG)],
        )

    return k(table, mpack, msrc, mdst, midx)


def kernel(z):
    mpack, msrc, mdst, midx = _build_meta(_IDX_NP)
    return _sc_scan_route(z, mpack, msrc, mdst, midx)


# speedup vs baseline: 1.2652x; 1.0741x over previous
import functools

import jax
import jax.numpy as jnp
import numpy as np
from jax import lax
from jax.experimental import pallas as pl
from jax.experimental.pallas import tpu as pltpu
from jax.experimental.pallas import tpu_sc as plsc

_N_WORDS = 8192
_N_SAMP = 65536
_WORD_DIM = 256
_NC = 2
_NS = 16
_CH = 16
_K = 8
_NBUF = 4
_XG = 176
_HALF = _N_WORDS // _NC


def _host_idx():
    cpu = jax.devices("cpu")[0]
    with jax.default_device(cpu):
        perm = jax.random.permutation(jax.random.key(1), _N_SAMP)
        return np.asarray(jax.device_get(perm))[:_N_WORDS].astype(np.int64)


_IDX_NP = _host_idx()


def _build_meta(idx_np):
    midx = np.zeros((_NC * _NS, _XG), np.int32)
    chunks = [{} for _ in range(_NC)]
    for i, src in enumerate(idx_np.tolist()):
        c = i // _HALF
        off = i % (_HALF // _NS)
        if off < _XG:
            midx[i // (_HALF // _NS), off] = src
            continue
        chunks[c].setdefault(src // _CH, []).append((src % _CH, i - c * _HALF))
    assign = {}
    max_ne = 0
    for c in range(_NC):
        order = sorted(chunks[c].items(), key=lambda kv: -len(kv[1]))
        loads = [(0, 0, t) for t in range(_NS)]
        lists = [[] for _ in range(_NS)]
        for cid, ent in order:
            assert len(ent) <= _K
            loads.sort()
            rows, ne, t = loads[0]
            lists[t].append((cid, ent))
            loads[0] = (rows + len(ent), ne + 1, t)
        for t in range(_NS):
            assign[c * _NS + t] = lists[t]
            max_ne = max(max_ne, len(lists[t]))
    lp = 2 + 2 * max_ne + 32
    ls = _K * max_ne + 32
    mpack = np.zeros((_NC * _NS, lp), np.int32)
    msrc = np.zeros((_NC * _NS, ls), np.int32)
    mdst = np.zeros((_NC * _NS, ls), np.int32)
    for w, lst in assign.items():
        mpack[w, 0] = len(lst)
        for e, (cid, ent) in enumerate(lst):
            mpack[w, 2 + 2 * e] = cid
            mpack[w, 3 + 2 * e] = len(ent)
            for j, (sl, dl) in enumerate(ent):
                msrc[w, _K * e + j] = sl
                mdst[w, _K * e + j] = dl
    return mpack, msrc, mdst, midx


def _sc_scan_route(table, mpack, msrc, mdst, midx):
    lp, ls = mpack.shape[1], msrc.shape[1]
    mesh = plsc.VectorSubcoreMesh(core_axis_name="c", subcore_axis_name="s")

    @functools.partial(
        pl.kernel,
        mesh=mesh,
        out_type=jax.ShapeDtypeStruct((_N_WORDS, _WORD_DIM), jnp.float32),
        scratch_types=[
            pltpu.VMEM((lp,), jnp.int32),
            pltpu.VMEM((ls,), jnp.int32),
            pltpu.VMEM((ls,), jnp.int32),
            pltpu.VMEM((_NBUF, _CH, _WORD_DIM), jnp.float32),
            pltpu.VMEM((1, _WORD_DIM), jnp.float32),
            pltpu.VMEM((_XG,), jnp.int32),
            pltpu.VMEM((_XG, _WORD_DIM), jnp.float32),
            pltpu.VMEM_SHARED((_HALF, _WORD_DIM), jnp.float32),
        ]
        + [pltpu.SemaphoreType.DMA] * (2 * _NBUF + 2),
    )
    def k(table_h, mpack_h, msrc_h, mdst_h, midx_h, out_h,
          mpack_v, msrc_v, mdst_v, bufs, drainbuf, midx_v, gbuf, spmem,
          *sems):
        ssem = sems[:_NBUF]
        rowsem = sems[_NBUF:2 * _NBUF]
        gsem = sems[2 * _NBUF]
        msem = sems[2 * _NBUF + 1]
        c = lax.axis_index("c")
        s = lax.axis_index("s")
        w = c * _NS + s
        mcopies = [
            pltpu.async_copy(midx_h.at[w], midx_v, msem),
            pltpu.async_copy(mpack_h.at[w], mpack_v, msem),
            pltpu.async_copy(msrc_h.at[w], msrc_v, msem),
            pltpu.async_copy(mdst_h.at[w], mdst_v, msem),
        ]
        mcopies[0].wait()
        gcopies = [
            pltpu.async_copy(
                table_h.at[midx_v.at[pl.ds(0, 128)]],
                gbuf.at[pl.ds(0, 128)], gsem),
            pltpu.async_copy(
                table_h.at[midx_v.at[pl.ds(128, _XG - 128)]],
                gbuf.at[pl.ds(128, _XG - 128)], gsem),
        ]
        for m in mcopies[1:]:
            m.wait()
        ne = mpack_v[pl.ds(0, 16)][0]

        def start(e, b):
            cid = mpack_v[pl.ds(2 + 2 * e, 16)][0]
            pltpu.async_copy(
                table_h.at[pl.ds(cid * _CH, _CH)], bufs.at[b], ssem[b]
            )

        for b in range(_NBUF - 1):
            @pl.when(b < ne)
            def _(b=b):
                start(b, b)

        @pl.loop(0, (ne + _NBUF) // _NBUF)
        def _(t):
            for b in range(_NBUF):
                g = t * _NBUF + b

                @pl.when(g < ne)
                def _(g=g, b=b):
                    pltpu.make_async_copy(
                        table_h.at[pl.ds(0, _CH)], bufs.at[b], ssem[b]
                    ).wait()
                    cnt = mpack_v[pl.ds(2 + 2 * g, 16)][1]
                    srow = msrc_v[pl.ds(_K * g, 16)]
                    drow = mdst_v[pl.ds(_K * g, 16)]
                    for j in range(_K):
                        @pl.when(j < cnt)
                        def _(j=j):
                            pltpu.async_copy(
                                bufs.at[b].at[pl.ds(srow[j], 1)],
                                spmem.at[pl.ds(drow[j], 1)],
                                rowsem[b],
                            )

                @pl.when((g >= 1) & (g <= ne))
                def _(g=g, b=b):
                    cp = mpack_v[pl.ds(2 * g, 16)][1]
                    for j in range(_K):
                        @pl.when(j < cp)
                        def _(j=j):
                            pltpu.make_async_copy(
                                table_h.at[pl.ds(0, 1)],
                                drainbuf,
                                rowsem[(b + _NBUF - 1) % _NBUF],
                            ).wait()

                @pl.when(g + (_NBUF - 1) < ne)
                def _(g=g, b=b):
                    start(g + (_NBUF - 1), (b + _NBUF - 1) % _NBUF)

        for gc in gcopies:
            gc.wait()
        base = c * _HALF + s * (_HALF // _NS)
        pltpu.sync_copy(gbuf, out_h.at[pl.ds(base, _XG)])
        plsc.subcore_barrier()
        pltpu.sync_copy(
            spmem.at[pl.ds(s * (_HALF // _NS) + _XG,
                           _HALF // _NS - _XG)],
            out_h.at[pl.ds(base + _XG, _HALF // _NS - _XG)],
        )

    return k(table, mpack, msrc, mdst, midx)


def kernel(z):
    mpack, msrc, mdst, midx = _build_meta(_IDX_NP)
    return _sc_scan_route(z, mpack, msrc, mdst, midx)


# speedup vs baseline: 1.5094x; 1.1929x over previous
import functools

import jax
import jax.numpy as jnp
import numpy as np
from jax import lax
from jax.experimental import pallas as pl
from jax.experimental.pallas import tpu as pltpu
from jax.experimental.pallas import tpu_sc as plsc

_N_WORDS = 8192
_N_SAMP = 65536
_WORD_DIM = 256
_NC = 2
_NS = 16
_CH = 16
_K = 8
_NBUF = 4
_XG = 208
_HALF = _N_WORDS // _NC
_TROWS = _HALF // _NS


def _host_idx():
    cpu = jax.devices("cpu")[0]
    with jax.default_device(cpu):
        perm = jax.random.permutation(jax.random.key(1), _N_SAMP)
        return np.asarray(jax.device_get(perm))[:_N_WORDS].astype(np.int64)


_IDX_NP = _host_idx()


def _build_meta(idx_np):
    midx = np.zeros((_NC * _NS, _XG), np.int32)
    chunks = [{} for _ in range(_NC)]
    for i, src in enumerate(idx_np.tolist()):
        c = i // _HALF
        off = i % (_HALF // _NS)
        if off < _XG:
            midx[i // (_HALF // _NS), off] = src
            continue
        dl = (i % _HALF) // _TROWS * (_TROWS - _XG) + (off - _XG)
        chunks[c].setdefault(src // _CH, []).append((src % _CH, dl))
    assign = {}
    max_ne = 0
    for c in range(_NC):
        order = sorted(chunks[c].items(), key=lambda kv: -len(kv[1]))
        loads = [(0, 0, t) for t in range(_NS)]
        lists = [[] for _ in range(_NS)]
        for cid, ent in order:
            assert len(ent) <= _K
            loads.sort()
            rows, ne, t = loads[0]
            lists[t].append((cid, ent))
            loads[0] = (rows + len(ent), ne + 1, t)
        for t in range(_NS):
            assign[c * _NS + t] = lists[t]
            max_ne = max(max_ne, len(lists[t]))
    lp = 2 + 2 * max_ne + 32
    ls = _K * max_ne + 32
    mpack = np.zeros((_NC * _NS, lp), np.int32)
    msrc = np.zeros((_NC * _NS, ls), np.int32)
    mdst = np.zeros((_NC * _NS, ls), np.int32)
    for w, lst in assign.items():
        mpack[w, 0] = len(lst)
        for e, (cid, ent) in enumerate(lst):
            mpack[w, 2 + 2 * e] = cid
            mpack[w, 3 + 2 * e] = len(ent)
            for j, (sl, dl) in enumerate(ent):
                msrc[w, _K * e + j] = sl
                mdst[w, _K * e + j] = dl
    return mpack, msrc, mdst, midx


def _sc_scan_route(table, mpack, msrc, mdst, midx):
    lp, ls = mpack.shape[1], msrc.shape[1]
    mesh = plsc.VectorSubcoreMesh(core_axis_name="c", subcore_axis_name="s")

    @functools.partial(
        pl.kernel,
        mesh=mesh,
        out_type=jax.ShapeDtypeStruct((_N_WORDS, _WORD_DIM), jnp.float32),
        scratch_types=[
            pltpu.VMEM((lp,), jnp.int32),
            pltpu.VMEM((ls,), jnp.int32),
            pltpu.VMEM((ls,), jnp.int32),
            pltpu.VMEM((_NBUF, _CH, _WORD_DIM), jnp.float32),
            pltpu.VMEM((1, _WORD_DIM), jnp.float32),
            pltpu.VMEM((_XG,), jnp.int32),
            pltpu.VMEM((_XG, _WORD_DIM), jnp.float32),
            pltpu.VMEM_SHARED((_NS * (_TROWS - _XG), _WORD_DIM), jnp.float32),
        ]
        + [pltpu.SemaphoreType.DMA] * (2 * _NBUF + 2),
    )
    def k(table_h, mpack_h, msrc_h, mdst_h, midx_h, out_h,
          mpack_v, msrc_v, mdst_v, bufs, drainbuf, midx_v, gbuf, spmem,
          *sems):
        ssem = sems[:_NBUF]
        rowsem = sems[_NBUF:2 * _NBUF]
        gsem = sems[2 * _NBUF]
        msem = sems[2 * _NBUF + 1]
        c = lax.axis_index("c")
        s = lax.axis_index("s")
        w = c * _NS + s
        mcopies = [
            pltpu.async_copy(midx_h.at[w], midx_v, msem),
            pltpu.async_copy(mpack_h.at[w], mpack_v, msem),
            pltpu.async_copy(msrc_h.at[w], msrc_v, msem),
            pltpu.async_copy(mdst_h.at[w], mdst_v, msem),
        ]
        mcopies[0].wait()
        gcopies = [
            pltpu.async_copy(
                table_h.at[midx_v.at[pl.ds(0, 128)]],
                gbuf.at[pl.ds(0, 128)], gsem),
            pltpu.async_copy(
                table_h.at[midx_v.at[pl.ds(128, _XG - 128)]],
                gbuf.at[pl.ds(128, _XG - 128)], gsem),
        ]
        for m in mcopies[1:]:
            m.wait()
        ne = mpack_v[pl.ds(0, 16)][0]

        def start(e, b):
            cid = mpack_v[pl.ds(2 + 2 * e, 16)][0]
            pltpu.async_copy(
                table_h.at[pl.ds(cid * _CH, _CH)], bufs.at[b], ssem[b]
            )

        for b in range(_NBUF - 1):
            @pl.when(b < ne)
            def _(b=b):
                start(b, b)

        @pl.loop(0, (ne + _NBUF) // _NBUF)
        def _(t):
            for b in range(_NBUF):
                g = t * _NBUF + b

                @pl.when(g < ne)
                def _(g=g, b=b):
                    pltpu.make_async_copy(
                        table_h.at[pl.ds(0, _CH)], bufs.at[b], ssem[b]
                    ).wait()
                    cnt = mpack_v[pl.ds(2 + 2 * g, 16)][1]
                    srow = msrc_v[pl.ds(_K * g, 16)]
                    drow = mdst_v[pl.ds(_K * g, 16)]
                    for j in range(_K):
                        @pl.when(j < cnt)
                        def _(j=j):
                            pltpu.async_copy(
                                bufs.at[b].at[pl.ds(srow[j], 1)],
                                spmem.at[pl.ds(drow[j], 1)],
                                rowsem[b],
                            )

                @pl.when((g >= 1) & (g <= ne))
                def _(g=g, b=b):
                    cp = mpack_v[pl.ds(2 * g, 16)][1]
                    for j in range(_K):
                        @pl.when(j < cp)
                        def _(j=j):
                            pltpu.make_async_copy(
                                table_h.at[pl.ds(0, 1)],
                                drainbuf,
                                rowsem[(b + _NBUF - 1) % _NBUF],
                            ).wait()

                @pl.when(g + (_NBUF - 1) < ne)
                def _(g=g, b=b):
                    start(g + (_NBUF - 1), (b + _NBUF - 1) % _NBUF)

        for gc in gcopies:
            gc.wait()
        base = c * _HALF + s * (_HALF // _NS)
        pltpu.sync_copy(gbuf, out_h.at[pl.ds(base, _XG)])
        plsc.subcore_barrier()
        pltpu.sync_copy(
            spmem.at[pl.ds(s * (_TROWS - _XG), _TROWS - _XG)],
            out_h.at[pl.ds(base + _XG, _TROWS - _XG)],
        )

    return k(table, mpack, msrc, mdst, midx)


def kernel(z):
    mpack, msrc, mdst, midx = _build_meta(_IDX_NP)
    return _sc_scan_route(z, mpack, msrc, mdst, midx)


# speedup vs baseline: 1.9026x; 1.2606x over previous
import functools

import jax
import jax.numpy as jnp
import numpy as np
from jax import lax
from jax.experimental import pallas as pl
from jax.experimental.pallas import tpu as pltpu
from jax.experimental.pallas import tpu_sc as plsc

_N_WORDS = 8192
_N_SAMP = 65536
_WORD_DIM = 256
_NC = 2
_NS = 16
_CH = 16
_K = 8
_NBUF = 4
_XG = 240
_HALF = _N_WORDS // _NC
_TROWS = _HALF // _NS


def _host_idx():
    cpu = jax.devices("cpu")[0]
    with jax.default_device(cpu):
        perm = jax.random.permutation(jax.random.key(1), _N_SAMP)
        return np.asarray(jax.device_get(perm))[:_N_WORDS].astype(np.int64)


_IDX_NP = _host_idx()


def _build_meta(idx_np):
    midx = np.zeros((_NC * _NS, _XG), np.int32)
    chunks = [{} for _ in range(_NC)]
    for i, src in enumerate(idx_np.tolist()):
        c = i // _HALF
        off = i % (_HALF // _NS)
        if off < _XG:
            midx[i // (_HALF // _NS), off] = src
            continue
        dl = (i % _HALF) // _TROWS * (_TROWS - _XG) + (off - _XG)
        chunks[c].setdefault(src // _CH, []).append((src % _CH, dl))
    assign = {}
    max_ne = 0
    for c in range(_NC):
        order = sorted(chunks[c].items(), key=lambda kv: -len(kv[1]))
        loads = [(0, 0, t) for t in range(_NS)]
        lists = [[] for _ in range(_NS)]
        for cid, ent in order:
            assert len(ent) <= _K
            loads.sort()
            rows, ne, t = loads[0]
            lists[t].append((cid, ent))
            loads[0] = (rows + len(ent), ne + 1, t)
        for t in range(_NS):
            assign[c * _NS + t] = lists[t]
            max_ne = max(max_ne, len(lists[t]))
    lp = 2 + 2 * max_ne + 32
    ls = _K * max_ne + 32
    mpack = np.zeros((_NC * _NS, lp), np.int32)
    msrc = np.zeros((_NC * _NS, ls), np.int32)
    mdst = np.zeros((_NC * _NS, ls), np.int32)
    for w, lst in assign.items():
        mpack[w, 0] = len(lst)
        for e, (cid, ent) in enumerate(lst):
            mpack[w, 2 + 2 * e] = cid
            mpack[w, 3 + 2 * e] = len(ent)
            for j, (sl, dl) in enumerate(ent):
                msrc[w, _K * e + j] = sl
                mdst[w, _K * e + j] = dl
    return mpack, msrc, mdst, midx


def _sc_scan_route(table, mpack, msrc, mdst, midx):
    lp, ls = mpack.shape[1], msrc.shape[1]
    mesh = plsc.VectorSubcoreMesh(core_axis_name="c", subcore_axis_name="s")

    @functools.partial(
        pl.kernel,
        mesh=mesh,
        out_type=jax.ShapeDtypeStruct((_N_WORDS, _WORD_DIM), jnp.float32),
        scratch_types=[
            pltpu.VMEM((lp,), jnp.int32),
            pltpu.VMEM((ls,), jnp.int32),
            pltpu.VMEM((ls,), jnp.int32),
            pltpu.VMEM((_NBUF, _CH, _WORD_DIM), jnp.float32),
            pltpu.VMEM((1, _WORD_DIM), jnp.float32),
            pltpu.VMEM((_XG,), jnp.int32),
            pltpu.VMEM((_XG, _WORD_DIM), jnp.float32),
            pltpu.VMEM_SHARED((_NS * (_TROWS - _XG), _WORD_DIM), jnp.float32),
        ]
        + [pltpu.SemaphoreType.DMA] * (2 * _NBUF + 2),
    )
    def k(table_h, mpack_h, msrc_h, mdst_h, midx_h, out_h,
          mpack_v, msrc_v, mdst_v, bufs, drainbuf, midx_v, gbuf, spmem,
          *sems):
        ssem = sems[:_NBUF]
        rowsem = sems[_NBUF:2 * _NBUF]
        gsem = sems[2 * _NBUF]
        msem = sems[2 * _NBUF + 1]
        c = lax.axis_index("c")
        s = lax.axis_index("s")
        w = c * _NS + s
        mcopies = [
            pltpu.async_copy(midx_h.at[w], midx_v, msem),
            pltpu.async_copy(mpack_h.at[w], mpack_v, msem),
            pltpu.async_copy(msrc_h.at[w], msrc_v, msem),
            pltpu.async_copy(mdst_h.at[w], mdst_v, msem),
        ]
        mcopies[0].wait()
        gcopies = [
            pltpu.async_copy(
                table_h.at[midx_v.at[pl.ds(0, 128)]],
                gbuf.at[pl.ds(0, 128)], gsem),
            pltpu.async_copy(
                table_h.at[midx_v.at[pl.ds(128, _XG - 128)]],
                gbuf.at[pl.ds(128, _XG - 128)], gsem),
        ]
        for m in mcopies[1:]:
            m.wait()
        ne = mpack_v[pl.ds(0, 16)][0]

        def start(e, b):
            cid = mpack_v[pl.ds(2 + 2 * e, 16)][0]
            pltpu.async_copy(
                table_h.at[pl.ds(cid * _CH, _CH)], bufs.at[b], ssem[b]
            )

        for b in range(_NBUF - 1):
            @pl.when(b < ne)
            def _(b=b):
                start(b, b)

        @pl.loop(0, (ne + _NBUF) // _NBUF)
        def _(t):
            for b in range(_NBUF):
                g = t * _NBUF + b

                @pl.when(g < ne)
                def _(g=g, b=b):
                    pltpu.make_async_copy(
                        table_h.at[pl.ds(0, _CH)], bufs.at[b], ssem[b]
                    ).wait()
                    cnt = mpack_v[pl.ds(2 + 2 * g, 16)][1]
                    srow = msrc_v[pl.ds(_K * g, 16)]
                    drow = mdst_v[pl.ds(_K * g, 16)]
                    for j in range(_K):
                        @pl.when(j < cnt)
                        def _(j=j):
                            pltpu.async_copy(
                                bufs.at[b].at[pl.ds(srow[j], 1)],
                                spmem.at[pl.ds(drow[j], 1)],
                                rowsem[b],
                            )

                @pl.when((g >= 1) & (g <= ne))
                def _(g=g, b=b):
                    cp = mpack_v[pl.ds(2 * g, 16)][1]
                    for j in range(_K):
                        @pl.when(j < cp)
                        def _(j=j):
                            pltpu.make_async_copy(
                                table_h.at[pl.ds(0, 1)],
                                drainbuf,
                                rowsem[(b + _NBUF - 1) % _NBUF],
                            ).wait()

                @pl.when(g + (_NBUF - 1) < ne)
                def _(g=g, b=b):
                    start(g + (_NBUF - 1), (b + _NBUF - 1) % _NBUF)

        for gc in gcopies:
            gc.wait()
        base = c * _HALF + s * (_HALF // _NS)
        pltpu.sync_copy(gbuf, out_h.at[pl.ds(base, _XG)])
        plsc.subcore_barrier()
        pltpu.sync_copy(
            spmem.at[pl.ds(s * (_TROWS - _XG), _TROWS - _XG)],
            out_h.at[pl.ds(base + _XG, _TROWS - _XG)],
        )

    return k(table, mpack, msrc, mdst, midx)


def kernel(z):
    mpack, msrc, mdst, midx = _build_meta(_IDX_NP)
    return _sc_scan_route(z, mpack, msrc, mdst, midx)


# speedup vs baseline: 2.0834x; 1.0950x over previous
import functools

import jax
import jax.numpy as jnp
import numpy as np
from jax import lax
from jax.experimental import pallas as pl
from jax.experimental.pallas import tpu as pltpu
from jax.experimental.pallas import tpu_sc as plsc

_N_WORDS = 8192
_N_SAMP = 65536
_WORD_DIM = 256
_NC = 2
_NS = 16
_CH = 16
_K = 8
_NBUF = 4
_XG = 248
_HALF = _N_WORDS // _NC
_TROWS = _HALF // _NS


def _host_idx():
    cpu = jax.devices("cpu")[0]
    with jax.default_device(cpu):
        perm = jax.random.permutation(jax.random.key(1), _N_SAMP)
        return np.asarray(jax.device_get(perm))[:_N_WORDS].astype(np.int64)


_IDX_NP = _host_idx()


def _build_meta(idx_np):
    midx = np.zeros((_NC * _NS, _XG), np.int32)
    chunks = [{} for _ in range(_NC)]
    for i, src in enumerate(idx_np.tolist()):
        c = i // _HALF
        off = i % (_HALF // _NS)
        if off < _XG:
            midx[i // (_HALF // _NS), off] = src
            continue
        dl = (i % _HALF) // _TROWS * (_TROWS - _XG) + (off - _XG)
        chunks[c].setdefault(src // _CH, []).append((src % _CH, dl))
    assign = {}
    max_ne = 0
    for c in range(_NC):
        order = sorted(chunks[c].items(), key=lambda kv: -len(kv[1]))
        loads = [(0, 0, t) for t in range(_NS)]
        lists = [[] for _ in range(_NS)]
        for cid, ent in order:
            assert len(ent) <= _K
            loads.sort()
            rows, ne, t = loads[0]
            lists[t].append((cid, ent))
            loads[0] = (rows + len(ent), ne + 1, t)
        for t in range(_NS):
            assign[c * _NS + t] = lists[t]
            max_ne = max(max_ne, len(lists[t]))
    lp = 2 + 2 * max_ne + 32
    ls = _K * max_ne + 32
    mpack = np.zeros((_NC * _NS, lp), np.int32)
    msrc = np.zeros((_NC * _NS, ls), np.int32)
    mdst = np.zeros((_NC * _NS, ls), np.int32)
    for w, lst in assign.items():
        mpack[w, 0] = len(lst)
        for e, (cid, ent) in enumerate(lst):
            mpack[w, 2 + 2 * e] = cid
            mpack[w, 3 + 2 * e] = len(ent)
            for j, (sl, dl) in enumerate(ent):
                msrc[w, _K * e + j] = sl
                mdst[w, _K * e + j] = dl
    return mpack, msrc, mdst, midx


def _sc_scan_route(table, mpack, msrc, mdst, midx):
    lp, ls = mpack.shape[1], msrc.shape[1]
    mesh = plsc.VectorSubcoreMesh(core_axis_name="c", subcore_axis_name="s")

    @functools.partial(
        pl.kernel,
        mesh=mesh,
        out_type=jax.ShapeDtypeStruct((_N_WORDS, _WORD_DIM), jnp.float32),
        scratch_types=[
            pltpu.VMEM((lp,), jnp.int32),
            pltpu.VMEM((ls,), jnp.int32),
            pltpu.VMEM((ls,), jnp.int32),
            pltpu.VMEM((_NBUF, _CH, _WORD_DIM), jnp.float32),
            pltpu.VMEM((1, _WORD_DIM), jnp.float32),
            pltpu.VMEM((_XG,), jnp.int32),
            pltpu.VMEM((_XG, _WORD_DIM), jnp.float32),
            pltpu.VMEM_SHARED((_NS * (_TROWS - _XG), _WORD_DIM), jnp.float32),
        ]
        + [pltpu.SemaphoreType.DMA] * (2 * _NBUF + 2),
    )
    def k(table_h, mpack_h, msrc_h, mdst_h, midx_h, out_h,
          mpack_v, msrc_v, mdst_v, bufs, drainbuf, midx_v, gbuf, spmem,
          *sems):
        ssem = sems[:_NBUF]
        rowsem = sems[_NBUF:2 * _NBUF]
        gsem = sems[2 * _NBUF]
        msem = sems[2 * _NBUF + 1]
        c = lax.axis_index("c")
        s = lax.axis_index("s")
        w = c * _NS + s
        mcopies = [
            pltpu.async_copy(midx_h.at[w], midx_v, msem),
            pltpu.async_copy(mpack_h.at[w], mpack_v, msem),
            pltpu.async_copy(msrc_h.at[w], msrc_v, msem),
            pltpu.async_copy(mdst_h.at[w], mdst_v, msem),
        ]
        mcopies[0].wait()
        gcopies = [
            pltpu.async_copy(
                table_h.at[midx_v.at[pl.ds(0, 128)]],
                gbuf.at[pl.ds(0, 128)], gsem),
            pltpu.async_copy(
                table_h.at[midx_v.at[pl.ds(128, _XG - 128)]],
                gbuf.at[pl.ds(128, _XG - 128)], gsem),
        ]
        for m in mcopies[1:]:
            m.wait()
        ne = mpack_v[pl.ds(0, 16)][0]

        def start(e, b):
            cid = mpack_v[pl.ds(2 + 2 * e, 16)][0]
            pltpu.async_copy(
                table_h.at[pl.ds(cid * _CH, _CH)], bufs.at[b], ssem[b]
            )

        for b in range(_NBUF - 1):
            @pl.when(b < ne)
            def _(b=b):
                start(b, b)

        @pl.loop(0, (ne + _NBUF) // _NBUF)
        def _(t):
            for b in range(_NBUF):
                g = t * _NBUF + b

                @pl.when(g < ne)
                def _(g=g, b=b):
                    pltpu.make_async_copy(
                        table_h.at[pl.ds(0, _CH)], bufs.at[b], ssem[b]
                    ).wait()
                    cnt = mpack_v[pl.ds(2 + 2 * g, 16)][1]
                    srow = msrc_v[pl.ds(_K * g, 16)]
                    drow = mdst_v[pl.ds(_K * g, 16)]
                    for j in range(_K):
                        @pl.when(j < cnt)
                        def _(j=j):
                            pltpu.async_copy(
                                bufs.at[b].at[pl.ds(srow[j], 1)],
                                spmem.at[pl.ds(drow[j], 1)],
                                rowsem[b],
                            )

                @pl.when((g >= 1) & (g <= ne))
                def _(g=g, b=b):
                    cp = mpack_v[pl.ds(2 * g, 16)][1]
                    for j in range(_K):
                        @pl.when(j < cp)
                        def _(j=j):
                            pltpu.make_async_copy(
                                table_h.at[pl.ds(0, 1)],
                                drainbuf,
                                rowsem[(b + _NBUF - 1) % _NBUF],
                            ).wait()

                @pl.when(g + (_NBUF - 1) < ne)
                def _(g=g, b=b):
                    start(g + (_NBUF - 1), (b + _NBUF - 1) % _NBUF)

        for gc in gcopies:
            gc.wait()
        base = c * _HALF + s * (_HALF // _NS)
        pltpu.sync_copy(gbuf, out_h.at[pl.ds(base, _XG)])
        plsc.subcore_barrier()
        pltpu.sync_copy(
            spmem.at[pl.ds(s * (_TROWS - _XG), _TROWS - _XG)],
            out_h.at[pl.ds(base + _XG, _TROWS - _XG)],
        )

    return k(table, mpack, msrc, mdst, midx)


def kernel(z):
    mpack, msrc, mdst, midx = _build_meta(_IDX_NP)
    return _sc_scan_route(z, mpack, msrc, mdst, midx)
